# baseline (speedup 1.0000x reference)
import functools

import jax
import jax.numpy as jnp
import numpy as np
from jax import lax
from jax.experimental import pallas as pl
from jax.experimental.pallas import tpu as pltpu
from jax.experimental.pallas import tpu_sc as plsc

B, P, N = 4, 16, 4096
S = 16
BP = B * P
CH = [2, 16, 32, 64, 64]
EPS = 1e-5
PACK = 1 << 18
_BN = float(1.0 / np.sqrt(1.0 + EPS))


def _selyz(y_dim):
    oy_dim = y_dim // 2
    sel1 = np.zeros((3, y_dim, oy_dim), np.float32)
    for d in range(3):
        for oy in range(oy_dim):
            y = 2 * oy + d - 1
            if 0 <= y < y_dim:
                sel1[d, y, oy] = 1.0
    out = np.einsum('dyo,ezp->deyzop', sel1, sel1)
    return out.reshape(9, y_dim * y_dim, oy_dim * oy_dim)


def _st_const(y_dim, co):
    s = _selyz(y_dim)
    return np.ascontiguousarray(
        np.broadcast_to(s[:, :, None, :], (9, s.shape[1], co, s.shape[2]))
    ).reshape(9, s.shape[1], co * s.shape[2])


def _rco_const(co, oyz):
    r = np.zeros((co, co * oyz), np.float32)
    for o in range(co):
        r[o, o * oyz:(o + 1) * oyz] = 1.0
    return r


_ST1 = _st_const(16, CH[1])
_ST2 = _st_const(8, CH[2])
_ST3 = _st_const(4, CH[3])
_ST4 = _st_const(2, CH[4])
_RCO1 = _rco_const(CH[1], 64)
_RCO2 = _rco_const(CH[2], 16)
_RCO3 = _rco_const(CH[3], 4)
_RCO4 = _rco_const(CH[4], 1)


def _build_bw(w27_ref, rco, st_ref, ci, y_dim):
    yz = y_dim * y_dim
    wexp = _mm(w27_ref[...], rco)
    c_dim = wexp.shape[1]
    wexp = wexp.reshape(27, ci, c_dim)
    bws = []
    for dx in range(3):
        acc = None
        for d in range(9):
            t = wexp[dx * 9 + d][:, None, :] * st_ref[d][None, :, :]
            acc = t if acc is None else acc + t
        bws.append(acc.reshape(ci * yz, c_dim))
    return jnp.stack(bws)


def _conv_mms(a, bw_ref):
    y = None
    for dx in range(3):
        t = _mm(a[dx], bw_ref[dx])
        y = t if y is None else y + t
    return y


def _xsel(x3, lanes):
    ox = x3.shape[1]
    even = x3[:, :, :lanes]
    odd = x3[:, :, lanes:]
    if ox > 1:
        shifted = jnp.concatenate(
            [jnp.zeros_like(odd[:, :1]), odd[:, :ox - 1]], axis=1)
    else:
        shifted = jnp.zeros_like(odd)
    m = BP * ox
    return (shifted.reshape(m, lanes), even.reshape(m, lanes),
            odd.reshape(m, lanes))


def _mm(a, b):
    return lax.dot_general(a, b, (((1,), (0,)), ((), ())),
                           preferred_element_type=jnp.float32)


def _bw_body(w1_ref, w2_ref, w3_ref, w4_ref,
             rco1_ref, rco2_ref, rco3_ref, rco4_ref,
             st1_ref, st2_ref, st3_ref, st4_ref,
             o1_ref, o2_ref, o3_ref, o4_ref):
    o1_ref[...] = _build_bw(w1_ref, rco1_ref[...], st1_ref, 2, 16)
    o2_ref[...] = _build_bw(w2_ref, rco2_ref[...], st2_ref, CH[1], 8)
    o3_ref[...] = _build_bw(w3_ref, rco3_ref[...], st3_ref, CH[2], 4)
    o4_ref[...] = _build_bw(w4_ref, rco4_ref[...], st4_ref, CH[3], 2)


def _encode_body(d_ref, bw1_ref, bw2_ref, bw3_ref, bw4_ref, wl_ref, o_ref):
    d = d_ref[...]
    cnt = (d >> 18).astype(jnp.float32)
    lbl = (d & (PACK - 1)).astype(jnp.float32)

    ac = _xsel(cnt.reshape(BP, 8, 512), 256)
    al = _xsel(lbl.reshape(BP, 8, 512), 256)
    a1 = tuple(jnp.concatenate([ac[i], al[i]], axis=1) for i in range(3))
    x = jnp.maximum(_conv_mms(a1, bw1_ref), 0.0).reshape(BP, 8, 1024)

    a = _xsel(x.reshape(BP, 4, 2048), 1024)
    x = jnp.maximum(_conv_mms(a, bw2_ref), 0.0).reshape(BP, 4, 512)

    a = _xsel(x.reshape(BP, 2, 1024), 512)
    x = jnp.maximum(_conv_mms(a, bw3_ref), 0.0).reshape(BP, 2, 256)

    a = _xsel(x.reshape(BP, 1, 512), 256)
    x = jnp.maximum(_conv_mms(a, bw4_ref), 0.0)

    o_ref[...] = lax.dot_general(x, wl_ref[...], (((1,), (1,)), ((), ())),
                                 preferred_element_type=jnp.float32)


@jax.jit
def _build_weights(w127, w227, w327, w427):
    return pl.pallas_call(
        _bw_body,
        out_shape=[
            jax.ShapeDtypeStruct((3, 512, 1024), jnp.float32),
            jax.ShapeDtypeStruct((3, 1024, 512), jnp.float32),
            jax.ShapeDtypeStruct((3, 512, 256), jnp.float32),
            jax.ShapeDtypeStruct((3, 256, 64), jnp.float32),
        ],
    )(w127, w227, w327, w427,
      jnp.asarray(_RCO1), jnp.asarray(_RCO2), jnp.asarray(_RCO3),
      jnp.asarray(_RCO4), jnp.asarray(_ST1), jnp.asarray(_ST2),
      jnp.asarray(_ST3), jnp.asarray(_ST4))


@functools.partial(jax.jit, static_argnames=("interpret",))
def _encode(dense_i32, bw1, bw2, bw3, bw4, W_lin, interpret=False):
    return pl.pallas_call(
        _encode_body,
        out_shape=jax.ShapeDtypeStruct((BP, CH[4]), jnp.float32),
        interpret=interpret,
    )(dense_i32, bw1, bw2, bw3, bw4, W_lin)


_SC_MESH = plsc.VectorSubcoreMesh(core_axis_name="c", subcore_axis_name="s")


@functools.partial(
    pl.kernel,
    mesh=_SC_MESH,
    out_type=jax.ShapeDtypeStruct((BP * N,), jnp.int32),
    scratch_types=[
        pltpu.VMEM((N,), jnp.int32),
        pltpu.VMEM((N,), jnp.int32),
        pltpu.VMEM((32, 128), jnp.int32),
        pltpu.VMEM((32, 128), jnp.int32),
        pltpu.VMEM((2 * N,), jnp.int32),
        pltpu.VMEM_SHARED((16 * 2 * N,), jnp.int32),
    ],
)
def _sc_scatter(gk_h, lb_h, out_h, kv, lv, iv, vv, zbuf, shared):
    cid = lax.axis_index("c")
    sid = lax.axis_index("s")
    wid = cid * 16 + sid
    reg = sid * (2 * N)

    def zbody(i, carry):
        zbuf[pl.ds(i * 16, 16)] = jnp.zeros((16,), jnp.int32)
        return carry

    lax.fori_loop(0, 512, zbody, 0)
    pltpu.sync_copy(zbuf, shared.at[pl.ds(reg, 2 * N)])

    for j in range(2):
        base = (wid * 2 + j) * N
        pltpu.sync_copy(gk_h.at[pl.ds(base, N)], kv)
        pltpu.sync_copy(lb_h.at[pl.ds(base, N)], lv)

        def cbody(k, carry):
            for u in range(8):
                o = (k * 8 + u) * 16
                iv[k, pl.ds(u * 16, 16)] = kv[pl.ds(o, 16)] + (reg + j * N)
                vv[k, pl.ds(u * 16, 16)] = lv[pl.ds(o, 16)] + PACK
            pltpu.sync_copy(vv.at[k], shared.at[iv.at[k]], add=True)
            return carry

        lax.fori_loop(0, 32, cbody, 0)

    pltpu.sync_copy(shared.at[pl.ds(reg, 2 * N)],
                    out_h.at[pl.ds(wid * 2 * N, 2 * N)])


def _prep_weights(w0, w1, w2, w3, g0, g1, g2, g3):
    w127 = (w0 * (g0 * _BN)).reshape(27 * CH[0], CH[1])
    w227 = (w1 * (g1 * _BN)).reshape(27 * CH[1], CH[2])
    w327 = (w2 * (g2 * _BN)).reshape(27 * CH[2], CH[3])
    w427 = (w3 * (g3 * _BN)).reshape(27 * CH[3], CH[4])
    return w127, w227, w327, w427


def kernel(parts_voxels, parts_labels, w0, w1, w2, w3, g0, g1, g2, g3, W_lin):
    coords = parts_voxels.reshape(BP * N, 3).astype(jnp.float32)
    gkf = coords @ jnp.array([[S * S], [S], [1.0]], jnp.float32)
    gk = gkf.reshape(BP * N).astype(jnp.int32)
    lb = parts_labels.reshape(BP * N)
    dense = _sc_scatter(gk, lb)
    dense = dense.reshape(BP, S, S * S)

    w127, w227, w327, w427 = _prep_weights(w0, w1, w2, w3, g0, g1, g2, g3)
    bw1, bw2, bw3, bw4 = _build_weights(w127, w227, w327, w427)
    out = _encode(dense, bw1, bw2, bw3, bw4, W_lin)
    return out.reshape(B, P, CH[4])


# speedup vs baseline: 1.3950x; 1.3950x over previous
import functools

import jax
import jax.numpy as jnp
import numpy as np
from jax import lax
from jax.experimental import pallas as pl
from jax.experimental.pallas import tpu as pltpu
from jax.experimental.pallas import tpu_sc as plsc

B, P, N = 4, 16, 4096
S = 16
BP = B * P
CH = [2, 16, 32, 64, 64]
EPS = 1e-5
PACK = 1 << 18
_BN = float(1.0 / np.sqrt(1.0 + EPS))


def _selyz(y_dim):
    oy_dim = y_dim // 2
    sel1 = np.zeros((3, y_dim, oy_dim), np.float32)
    for d in range(3):
        for oy in range(oy_dim):
            y = 2 * oy + d - 1
            if 0 <= y < y_dim:
                sel1[d, y, oy] = 1.0
    out = np.einsum('dyo,ezp->deyzop', sel1, sel1)
    return out.reshape(9, y_dim * y_dim, oy_dim * oy_dim)


def _st_const(y_dim, co):
    s = _selyz(y_dim)
    return np.ascontiguousarray(
        np.broadcast_to(s[:, :, None, :], (9, s.shape[1], co, s.shape[2]))
    ).reshape(9, s.shape[1], co * s.shape[2])


def _rco_const(co, oyz):
    r = np.zeros((co, co * oyz), np.float32)
    for o in range(co):
        r[o, o * oyz:(o + 1) * oyz] = 1.0
    return r


_ST1 = _st_const(16, CH[1])
_ST2 = _st_const(8, CH[2])
_ST3 = _st_const(4, CH[3])
_ST4 = _st_const(2, CH[4])
_RCO1 = _rco_const(CH[1], 64)
_RCO2 = _rco_const(CH[2], 16)
_RCO3 = _rco_const(CH[3], 4)
_RCO4 = _rco_const(CH[4], 1)


def _build_bw(w27_ref, g_ref, rco, st_ref, ci, y_dim):
    yz = y_dim * y_dim
    w = w27_ref[...] * (g_ref[...] * _BN)
    wexp = _mm(w, rco).astype(jnp.bfloat16)
    c_dim = wexp.shape[1]
    wexp = wexp.reshape(27, ci, c_dim)
    bws = []
    for dx in range(3):
        acc = None
        for d in range(9):
            t = wexp[dx * 9 + d][:, None, :] * st_ref[d][None, :, :]
            acc = t if acc is None else acc + t
        bws.append(acc.reshape(ci * yz, c_dim))
    return jnp.stack(bws)


def _conv_mms(a, bw_ref):
    y = None
    for dx in range(3):
        t = _mm(a[dx], bw_ref[dx].astype(jnp.float32))
        y = t if y is None else y + t
    return y


def _xsel(x3, lanes):
    ox = x3.shape[1]
    even = x3[:, :, :lanes]
    odd = x3[:, :, lanes:]
    if ox > 1:
        shifted = jnp.concatenate(
            [jnp.zeros_like(odd[:, :1]), odd[:, :ox - 1]], axis=1)
    else:
        shifted = jnp.zeros_like(odd)
    m = BP * ox
    return (shifted.reshape(m, lanes), even.reshape(m, lanes),
            odd.reshape(m, lanes))


def _mm(a, b):
    return lax.dot_general(a, b, (((1,), (0,)), ((), ())),
                           preferred_element_type=jnp.float32)


def _bw_body(w1_ref, w2_ref, w3_ref, w4_ref,
             g1_ref, g2_ref, g3_ref, g4_ref,
             rco1_ref, rco2_ref, rco3_ref, rco4_ref,
             st1_ref, st2_ref, st3_ref, st4_ref,
             o1_ref, o2_ref, o3_ref, o4_ref):
    o1_ref[...] = _build_bw(w1_ref, g1_ref, rco1_ref[...], st1_ref, 2, 16)
    o2_ref[...] = _build_bw(w2_ref, g2_ref, rco2_ref[...], st2_ref, CH[1], 8)
    o3_ref[...] = _build_bw(w3_ref, g3_ref, rco3_ref[...], st3_ref, CH[2], 4)
    o4_ref[...] = _build_bw(w4_ref, g4_ref, rco4_ref[...], st4_ref, CH[3], 2)


def _encode_body(d_ref, bw1_ref, bw2_ref, bw3_ref, bw4_ref, wl_ref, o_ref):
    d = d_ref[...]
    cnt = (d >> 18).astype(jnp.float32)
    lbl = (d & (PACK - 1)).astype(jnp.float32)

    ac = _xsel(cnt.reshape(BP, 8, 512), 256)
    al = _xsel(lbl.reshape(BP, 8, 512), 256)
    a1 = tuple(jnp.concatenate([ac[i], al[i]], axis=1) for i in range(3))
    x = jnp.maximum(_conv_mms(a1, bw1_ref), 0.0).reshape(BP, 8, 1024)

    a = _xsel(x.reshape(BP, 4, 2048), 1024)
    x = jnp.maximum(_conv_mms(a, bw2_ref), 0.0).reshape(BP, 4, 512)

    a = _xsel(x.reshape(BP, 2, 1024), 512)
    x = jnp.maximum(_conv_mms(a, bw3_ref), 0.0).reshape(BP, 2, 256)

    a = _xsel(x.reshape(BP, 1, 512), 256)
    x = jnp.maximum(_conv_mms(a, bw4_ref), 0.0)

    o_ref[...] = lax.dot_general(x, wl_ref[...], (((1,), (1,)), ((), ())),
                                 preferred_element_type=jnp.float32)


@jax.jit
def _build_weights(w127, w227, w327, w427, g0, g1, g2, g3):
    return pl.pallas_call(
        _bw_body,
        out_shape=[
            jax.ShapeDtypeStruct((3, 512, 1024), jnp.bfloat16),
            jax.ShapeDtypeStruct((3, 1024, 512), jnp.bfloat16),
            jax.ShapeDtypeStruct((3, 512, 256), jnp.bfloat16),
            jax.ShapeDtypeStruct((3, 256, 64), jnp.bfloat16),
        ],
    )(w127, w227, w327, w427, g0, g1, g2, g3,
      jnp.asarray(_RCO1), jnp.asarray(_RCO2), jnp.asarray(_RCO3),
      jnp.asarray(_RCO4),
      jnp.asarray(_ST1, jnp.bfloat16), jnp.asarray(_ST2, jnp.bfloat16),
      jnp.asarray(_ST3, jnp.bfloat16), jnp.asarray(_ST4, jnp.bfloat16))


@functools.partial(jax.jit, static_argnames=("interpret",))
def _encode(dense_i32, bw1, bw2, bw3, bw4, W_lin, interpret=False):
    return pl.pallas_call(
        _encode_body,
        out_shape=jax.ShapeDtypeStruct((BP, CH[4]), jnp.float32),
        interpret=interpret,
    )(dense_i32, bw1, bw2, bw3, bw4, W_lin)


_SC_MESH = plsc.VectorSubcoreMesh(core_axis_name="c", subcore_axis_name="s")


@functools.partial(
    pl.kernel,
    mesh=_SC_MESH,
    out_type=jax.ShapeDtypeStruct((BP * N,), jnp.int32),
    scratch_types=[
        pltpu.VMEM((N,), jnp.float32),
        pltpu.VMEM((N,), jnp.int32),
        pltpu.VMEM((32, 128), jnp.int32),
        pltpu.VMEM((32, 128), jnp.int32),
        pltpu.VMEM((2 * N,), jnp.int32),
        pltpu.VMEM_SHARED((16 * 2 * N,), jnp.int32),
    ],
)
def _sc_scatter(gk_h, lb_h, out_h, kv, lv, iv, vv, zbuf, shared):
    cid = lax.axis_index("c")
    sid = lax.axis_index("s")
    wid = cid * 16 + sid
    reg = sid * (2 * N)

    def zbody(i, carry):
        zbuf[pl.ds(i * 16, 16)] = jnp.zeros((16,), jnp.int32)
        return carry

    lax.fori_loop(0, 512, zbody, 0)
    pltpu.sync_copy(zbuf, shared.at[pl.ds(reg, 2 * N)])

    for j in range(2):
        base = (wid * 2 + j) * N
        pltpu.sync_copy(gk_h.at[pl.ds(base, N)], kv)
        pltpu.sync_copy(lb_h.at[pl.ds(base, N)], lv)

        def cbody(k, carry):
            for u in range(8):
                o = (k * 8 + u) * 16
                ki = kv[pl.ds(o, 16)].astype(jnp.int32)
                iv[k, pl.ds(u * 16, 16)] = ki + (reg + j * N)
                vv[k, pl.ds(u * 16, 16)] = lv[pl.ds(o, 16)] + PACK
            pltpu.sync_copy(vv.at[k], shared.at[iv.at[k]], add=True)
            return carry

        lax.fori_loop(0, 32, cbody, 0)

    pltpu.sync_copy(shared.at[pl.ds(reg, 2 * N)],
                    out_h.at[pl.ds(wid * 2 * N, 2 * N)])


def _prep_weights(w0, w1, w2, w3):
    return (w0.reshape(27 * CH[0], CH[1]), w1.reshape(27 * CH[1], CH[2]),
            w2.reshape(27 * CH[2], CH[3]), w3.reshape(27 * CH[3], CH[4]))


def kernel(parts_voxels, parts_labels, w0, w1, w2, w3, g0, g1, g2, g3, W_lin):
    coords = parts_voxels.reshape(BP * N, 3).astype(jnp.float32)
    gkf = coords @ jnp.array([[S * S], [S], [1.0]], jnp.float32)
    gk = gkf.reshape(BP * N)
    lb = parts_labels.reshape(BP * N)
    dense = _sc_scatter(gk, lb)
    dense = dense.reshape(BP, S, S * S)

    w127, w227, w327, w427 = _prep_weights(w0, w1, w2, w3)
    bw1, bw2, bw3, bw4 = _build_weights(w127, w227, w327, w427,
                                        g0, g1, g2, g3)
    out = _encode(dense, bw1, bw2, bw3, bw4, W_lin)
    return out.reshape(B, P, CH[4])
